# baseline (device time: 13557 ns/iter reference)
import jax
import jax.numpy as jnp
from jax import lax
from jax.experimental import pallas as pl
from jax.experimental.pallas import tpu as pltpu

N_DEV = 4
BLK = 64


def kernel(x, Wq, K_ext, V_ext, Wo):
    B, Sq_l, D = x.shape
    _, Skv_l, Hq, Dh = K_ext.shape
    n_qblk = Sq_l // BLK

    def body(x_ref, wq_ref, k_ref, v_ref, wo_ref, out_ref,
             kv16_ref, kv8_ref, kvrecv_ref, vrecv16_ref, ctx_ref, wr16_ref,
             send_sem, recv_sem):
        my = lax.axis_index("i")
        partner = (my + 2) % N_DEV

        barrier_sem = pltpu.get_barrier_semaphore()
        pl.semaphore_signal(
            barrier_sem, inc=1,
            device_id=(partner,), device_id_type=pl.DeviceIdType.MESH,
        )

        kv8_ref[...] = k_ref[...].astype(jnp.float8_e4m3fn)
        kv16_ref[0] = k_ref[...].astype(jnp.bfloat16)
        kv16_ref[1] = v_ref[...].astype(jnp.bfloat16)

        pl.semaphore_wait(barrier_sem, 1)

        rdma_k = pltpu.make_async_remote_copy(
            src_ref=kv8_ref, dst_ref=kvrecv_ref,
            send_sem=send_sem.at[0], recv_sem=recv_sem.at[0],
            device_id=(partner,), device_id_type=pl.DeviceIdType.MESH,
        )
        rdma_v = pltpu.make_async_remote_copy(
            src_ref=kv16_ref.at[1], dst_ref=vrecv16_ref,
            send_sem=send_sem.at[1], recv_sem=recv_sem.at[1],
            device_id=(partner,), device_id_type=pl.DeviceIdType.MESH,
        )
        rdma_k.start()
        rdma_v.start()

        wq16 = wq_ref[...].astype(jnp.bfloat16)
        q16 = [
            jnp.dot(
                x_ref[b].astype(jnp.bfloat16), wq16,
                preferred_element_type=jnp.float32,
            ).astype(jnp.bfloat16)
            for b in range(B)
        ]

        rows = lax.broadcasted_iota(jnp.int32, (Sq_l, Skv_l), 0)
        cols = lax.broadcasted_iota(jnp.int32, (Sq_l, Skv_l), 1)
        maskf = (cols // BLK == rows // BLK).astype(jnp.float32)

        dn = (((1,), (1,)), ((), ()))
        sums = [[None] * Hq for _ in range(B)]
        for b in range(B):
            for h in range(Hq):
                c0, c1 = h * Dh, (h + 1) * Dh
                sl = lax.dot_general(
                    q16[b][:, c0:c1], kv16_ref[0, b, :, h, :], dn,
                    preferred_element_type=jnp.float32,
                ) * 0.125
                wl = jnp.exp(sl) * maskf
                sums[b][h] = wl.sum(axis=-1, keepdims=True)
                ctx_ref[b, :, c0:c1] = jnp.dot(
                    wl.astype(jnp.bfloat16), kv16_ref[1, b, :, h, :],
                    preferred_element_type=jnp.float32,
                )

        rdma_k.wait_recv()

        wo16 = wo_ref[...].astype(jnp.bfloat16)
        inv = [[None] * Hq for _ in range(B)]
        for b in range(B):
            for h in range(Hq):
                c0, c1 = h * Dh, (h + 1) * Dh
                sr = lax.dot_general(
                    q16[b][:, c0:c1],
                    kvrecv_ref[b, :, h, :].astype(jnp.bfloat16), dn,
                    preferred_element_type=jnp.float32,
                ) * 0.125
                wr = jnp.exp(sr) * maskf
                inv[b][h] = 1.0 / (sums[b][h] + wr.sum(axis=-1, keepdims=True))
                wr16_ref[b, h] = wr.astype(jnp.bfloat16)

        rdma_v.wait_recv()

        for b in range(B):
            ctx16_cols = []
            for h in range(Hq):
                c0, c1 = h * Dh, (h + 1) * Dh
                ctx = ctx_ref[b, :, c0:c1] + jnp.dot(
                    wr16_ref[b, h],
                    vrecv16_ref[b, :, h, :],
                    preferred_element_type=jnp.float32,
                )
                ctx16_cols.append((ctx * inv[b][h]).astype(jnp.bfloat16))
            ctx16 = jnp.concatenate(ctx16_cols, axis=1)
            out_ref[b] = jnp.dot(
                ctx16, wo16, preferred_element_type=jnp.float32
            )

        rdma_k.wait_send()
        rdma_v.wait_send()

    return pl.pallas_call(
        body,
        out_shape=jax.ShapeDtypeStruct((B, Sq_l, D), jnp.float32),
        in_specs=[pl.BlockSpec(memory_space=pltpu.VMEM)] * 5,
        out_specs=pl.BlockSpec(memory_space=pltpu.VMEM),
        scratch_shapes=[
            pltpu.VMEM((2, B, Skv_l, Hq, Dh), jnp.bfloat16),
            pltpu.VMEM((B, Skv_l, Hq, Dh), jnp.float8_e4m3fn),
            pltpu.VMEM((B, Skv_l, Hq, Dh), jnp.float8_e4m3fn),
            pltpu.VMEM((B, Skv_l, Hq, Dh), jnp.bfloat16),
            pltpu.VMEM((B, Sq_l, Hq * Dh), jnp.float32),
            pltpu.VMEM((B, Hq, Sq_l, Skv_l), jnp.bfloat16),
            pltpu.SemaphoreType.DMA((2,)),
            pltpu.SemaphoreType.DMA((2,)),
        ],
        compiler_params=pltpu.CompilerParams(collective_id=0),
    )(x, Wq, K_ext, V_ext, Wo)


# device time: 13164 ns/iter; 1.0299x vs baseline; 1.0299x over previous
import jax
import jax.numpy as jnp
from jax import lax
from jax.experimental import pallas as pl
from jax.experimental.pallas import tpu as pltpu

N_DEV = 4
BLK = 64


def kernel(x, Wq, K_ext, V_ext, Wo):
    B, Sq_l, D = x.shape
    _, Skv_l, Hq, Dh = K_ext.shape
    n_qblk = Sq_l // BLK

    def body(x_ref, wq_ref, k_ref, v_ref, wo_ref, out_ref,
             kv16_ref, kv8_ref, kvrecv_ref, vrecv16_ref, ctx_ref, wr16_ref,
             send_sem, recv_sem):
        my = lax.axis_index("i")
        partner = (my + 2) % N_DEV

        barrier_sem = pltpu.get_barrier_semaphore()
        pl.semaphore_signal(
            barrier_sem, inc=1,
            device_id=(partner,), device_id_type=pl.DeviceIdType.MESH,
        )

        kv8_ref[...] = k_ref[...].astype(jnp.float8_e4m3fn)
        kv16_ref[0] = k_ref[...].astype(jnp.bfloat16)
        kv16_ref[1] = v_ref[...].astype(jnp.bfloat16)

        pl.semaphore_wait(barrier_sem, 1)

        rdma_k = pltpu.make_async_remote_copy(
            src_ref=kv8_ref, dst_ref=kvrecv_ref,
            send_sem=send_sem.at[0], recv_sem=recv_sem.at[0],
            device_id=(partner,), device_id_type=pl.DeviceIdType.MESH,
        )
        rdma_v0 = pltpu.make_async_remote_copy(
            src_ref=kv16_ref.at[1, 0], dst_ref=vrecv16_ref.at[0],
            send_sem=send_sem.at[1], recv_sem=recv_sem.at[1],
            device_id=(partner,), device_id_type=pl.DeviceIdType.MESH,
        )
        rdma_v1 = pltpu.make_async_remote_copy(
            src_ref=kv16_ref.at[1, 1], dst_ref=vrecv16_ref.at[1],
            send_sem=send_sem.at[2], recv_sem=recv_sem.at[2],
            device_id=(partner,), device_id_type=pl.DeviceIdType.MESH,
        )
        rdma_k.start()
        rdma_v0.start()
        rdma_v1.start()

        wq16 = wq_ref[...].astype(jnp.bfloat16)
        q16 = [
            jnp.dot(
                x_ref[b].astype(jnp.bfloat16), wq16,
                preferred_element_type=jnp.float32,
            ).astype(jnp.bfloat16)
            for b in range(B)
        ]

        rows = lax.broadcasted_iota(jnp.int32, (Sq_l, Skv_l), 0)
        cols = lax.broadcasted_iota(jnp.int32, (Sq_l, Skv_l), 1)
        maskf = (cols // BLK == rows // BLK).astype(jnp.float32)

        dn = (((1,), (1,)), ((), ()))
        sums = [[None] * Hq for _ in range(B)]
        for b in range(B):
            for h in range(Hq):
                c0, c1 = h * Dh, (h + 1) * Dh
                sl = lax.dot_general(
                    q16[b][:, c0:c1], kv16_ref[0, b, :, h, :], dn,
                    preferred_element_type=jnp.float32,
                ) * 0.125
                wl = jnp.exp(sl) * maskf
                sums[b][h] = wl.sum(axis=-1, keepdims=True)
                ctx_ref[b, :, c0:c1] = jnp.dot(
                    wl.astype(jnp.bfloat16), kv16_ref[1, b, :, h, :],
                    preferred_element_type=jnp.float32,
                )

        rdma_k.wait_recv()

        wo16 = wo_ref[...].astype(jnp.bfloat16)
        inv = [[None] * Hq for _ in range(B)]
        for b in range(B):
            for h in range(Hq):
                c0, c1 = h * Dh, (h + 1) * Dh
                sr = lax.dot_general(
                    q16[b][:, c0:c1],
                    kvrecv_ref[b, :, h, :].astype(jnp.bfloat16), dn,
                    preferred_element_type=jnp.float32,
                ) * 0.125
                wr = jnp.exp(sr) * maskf
                inv[b][h] = 1.0 / (sums[b][h] + wr.sum(axis=-1, keepdims=True))
                wr16_ref[b, h] = wr.astype(jnp.bfloat16)

        for b in range(B):
            (rdma_v0 if b == 0 else rdma_v1).wait_recv()
            ctx16_cols = []
            for h in range(Hq):
                c0, c1 = h * Dh, (h + 1) * Dh
                ctx = ctx_ref[b, :, c0:c1] + jnp.dot(
                    wr16_ref[b, h],
                    vrecv16_ref[b, :, h, :],
                    preferred_element_type=jnp.float32,
                )
                ctx16_cols.append((ctx * inv[b][h]).astype(jnp.bfloat16))
            ctx16 = jnp.concatenate(ctx16_cols, axis=1)
            out_ref[b] = jnp.dot(
                ctx16, wo16, preferred_element_type=jnp.float32
            )

        rdma_k.wait_send()
        rdma_v0.wait_send()
        rdma_v1.wait_send()

    return pl.pallas_call(
        body,
        out_shape=jax.ShapeDtypeStruct((B, Sq_l, D), jnp.float32),
        in_specs=[pl.BlockSpec(memory_space=pltpu.VMEM)] * 5,
        out_specs=pl.BlockSpec(memory_space=pltpu.VMEM),
        scratch_shapes=[
            pltpu.VMEM((2, B, Skv_l, Hq, Dh), jnp.bfloat16),
            pltpu.VMEM((B, Skv_l, Hq, Dh), jnp.float8_e4m3fn),
            pltpu.VMEM((B, Skv_l, Hq, Dh), jnp.float8_e4m3fn),
            pltpu.VMEM((B, Skv_l, Hq, Dh), jnp.bfloat16),
            pltpu.VMEM((B, Sq_l, Hq * Dh), jnp.float32),
            pltpu.VMEM((B, Hq, Sq_l, Skv_l), jnp.bfloat16),
            pltpu.SemaphoreType.DMA((3,)),
            pltpu.SemaphoreType.DMA((3,)),
        ],
        compiler_params=pltpu.CompilerParams(collective_id=0),
    )(x, Wq, K_ext, V_ext, Wo)
